# Initial kernel scaffold; baseline (speedup 1.0000x reference)
import jax
import jax.numpy as jnp
from jax.experimental import pallas as pl


def kernel(nodes, mask, table):
    raise NotImplementedError("write your pallas kernel here")


# speedup vs baseline: 1.3125x; 1.3125x over previous
import functools

import jax
import jax.numpy as jnp
from jax import lax
from jax.experimental import pallas as pl
from jax.experimental.pallas import tpu as pltpu
from jax.experimental.pallas import tpu_sc as plsc

_NC = 2
_NS = 16
_NW = _NC * _NS
_LANES = 16

_G = 8
_ROWS_PER_DMA = 128
_CHUNK = _G * _ROWS_PER_DMA


def _body(nodes_hbm, mask_hbm, table_hbm, out_hbm, idx_v, msk_v, rows_v, sem):
    n_groups = nodes_hbm.shape[0]
    d = table_hbm.shape[1]
    groups_per_w = n_groups // _NW
    chunks_per_w = groups_per_w // _G

    wid = lax.axis_index("s") * _NC + lax.axis_index("c")
    w_group0 = wid * groups_per_w

    zeros16 = jnp.zeros((_LANES,), jnp.float32)
    iota16 = lax.iota(jnp.int32, _LANES)

    def chunk_body(t, carry):
        bg = w_group0 + t * _G
        pltpu.sync_copy(nodes_hbm.at[pl.ds(bg, _G)], idx_v)
        pltpu.sync_copy(mask_hbm.at[pl.ds(bg, _G)], msk_v)
        copies = [
            pltpu.async_copy(
                table_hbm.at[idx_v.at[g]], rows_v.at[pl.ds(g * _ROWS_PER_DMA, _ROWS_PER_DMA)], sem
            )
            for g in range(_G)
        ]
        for c in copies:
            c.wait()

        def strip_body(s, carry2):
            g = s // (_ROWS_PER_DMA // _LANES)
            r0 = (s % (_ROWS_PER_DMA // _LANES)) * _LANES
            mv = msk_v[g, pl.ds(r0, _LANES)]
            invalid = mv == 0
            rowvec = iota16 + s * _LANES
            cv = jnp.zeros((_LANES,), jnp.int32)
            for _c in range(d):
                plsc.store_scatter(rows_v, [rowvec, cv], zeros16, mask=invalid)
                cv = cv + 1
            return carry2

        lax.fori_loop(0, _CHUNK // _LANES, strip_body, 0, unroll=False)

        pltpu.sync_copy(rows_v, out_hbm.at[pl.ds(bg * _ROWS_PER_DMA, _CHUNK)])
        return carry

    lax.fori_loop(0, chunks_per_w, chunk_body, 0, unroll=False)


@functools.partial(jax.jit, static_argnames=())
def _sc_lookup(nodes2d, mask2d, table):
    n_groups = nodes2d.shape[0]
    n_rows = n_groups * _ROWS_PER_DMA
    d = table.shape[1]
    mesh = plsc.VectorSubcoreMesh(core_axis_name="c", subcore_axis_name="s")
    return pl.kernel(
        _body,
        out_type=jax.ShapeDtypeStruct((n_rows, d), jnp.float32),
        mesh=mesh,
        scratch_types=[
            pltpu.VMEM((_G, _ROWS_PER_DMA), jnp.int32),
            pltpu.VMEM((_G, _ROWS_PER_DMA), jnp.int32),
            pltpu.VMEM((_CHUNK, d), jnp.float32),
            pltpu.SemaphoreType.DMA,
        ],
        compiler_params=pltpu.CompilerParams(
            needs_layout_passes=False,
            use_tc_tiling_on_sc=False,
        ),
        name="masked_embedding_gather",
    )(nodes2d, mask2d, table)


def kernel(nodes, mask, table):
    b, l = nodes.shape
    v, d = table.shape
    n = b * l
    assert n % (_NW * _CHUNK) == 0
    nodes2d = nodes.reshape(n // _ROWS_PER_DMA, _ROWS_PER_DMA)
    mask2d = mask.astype(jnp.int32).reshape(n // _ROWS_PER_DMA, _ROWS_PER_DMA)
    out = _sc_lookup(nodes2d, mask2d, table)
    return out.reshape(b, l, d)
